# baseline (speedup 1.0000x reference)
import functools

import jax
import jax.numpy as jnp
from jax.experimental import pallas as pl
from jax.experimental.pallas import tpu as pltpu


def _arb_loss_body(out_ref, y_ref, loss_ref, acc_ref, cnt_ref):
    i = pl.program_id(0)
    nblk = pl.num_programs(0)
    blk, C = out_ref.shape
    n = blk * nblk

    @pl.when(i == 0)
    def _init():
        acc_ref[0, 0] = jnp.float32(0.0)
        cnt_ref[...] = jnp.zeros_like(cnt_ref)

    x = out_ref[...]
    s = jnp.sum(x, axis=1, keepdims=True)
    acc_ref[0, 0] += jnp.sum(jnp.log(s))

    @pl.when(i == nblk - 1)
    def _fini():
        cnt = cnt_ref[...]
        cterm = jnp.sum(cnt * jnp.log(jnp.maximum(cnt, 1.0)))
        loss_ref[0, 0] = (jnp.log(jnp.float32(n))
                          + (acc_ref[0, 0] - cterm) / jnp.float32(n))


@functools.partial(jax.jit, static_argnames=("blk",))
def _arb_loss(output, y, blk=2048):
    n, C = output.shape
    output = output.reshape(16000, 1024)
    n, C = output.shape
    y2 = y.astype(jnp.int32)[:16000].reshape(n, 1)
    grid = n // blk
    out = pl.pallas_call(
        _arb_loss_body,
        grid=(grid,),
        in_specs=[
            pl.BlockSpec((blk, C), lambda i: (i, 0)),
            pl.BlockSpec((blk, 1), lambda i: (i, 0)),
        ],
        out_specs=pl.BlockSpec(memory_space=pltpu.SMEM),
        out_shape=jax.ShapeDtypeStruct((1, 1), jnp.float32),
        scratch_shapes=[
            pltpu.SMEM((1, 1), jnp.float32),
            pltpu.VMEM((1, C), jnp.float32),
        ],
        compiler_params=pltpu.CompilerParams(
            dimension_semantics=("arbitrary",),
        ),
    )(output, y2)
    return out.reshape(())


def kernel(output, y):
    return _arb_loss(output, y)


# speedup vs baseline: 1.2850x; 1.2850x over previous
import functools

import jax
import jax.numpy as jnp
from jax import lax
from jax.experimental import pallas as pl
from jax.experimental.pallas import tpu as pltpu
from jax.experimental.pallas import tpu_sc as plsc

_N = 16384
_C = 1000
_NW = 32
_RPW = _N // _NW
_CH = 32
_NCHUNK = _RPW // _CH
_HB = 1008


def _sc_body(out_hbm, y_hbm, sp_hbm, picked_hbm, hist_hbm,
             y_v, buf_a, buf_b, acc_v, picked_v, hist_v, red_v, sem_a, sem_b):
    cid = lax.axis_index("c")
    sid = lax.axis_index("s")
    wid = sid * 2 + cid
    base = wid * _RPW

    lane = lax.broadcasted_iota(jnp.int32, (16,), 0)
    lane_off = lane * _HB
    tail_mask = lane >= 8
    zeros16 = jnp.zeros((16,), jnp.float32)
    ones16i = jnp.ones((16,), jnp.int32)

    pltpu.sync_copy(y_hbm.at[pl.ds(base, _RPW)], y_v)

    def _zero(i, _):
        hist_v[pl.ds(i * 16, 16)] = jnp.zeros((16,), jnp.int32)
        return 0
    lax.fori_loop(0, (16 * _HB) // 16, _zero, 0)

    def _issue(ck, buf, sem):
        pltpu.async_copy(out_hbm.at[pl.ds(base + ck * _CH, _CH)], buf, sem)

    def _drain(buf, sem):
        pltpu.make_async_copy(out_hbm.at[pl.ds(base, _CH)], buf, sem).wait()

    def _process(ck, buf):
        def _row(r, _):
            rt = ck * _CH + r
            total = zeros16
            for j in range(_C // 16):
                total = total + buf[r, pl.ds(j * 16, 16)]
            tail = buf[r, pl.ds(_C - 16, 16)]
            total = total + jnp.where(tail_mask, tail, zeros16)
            acc_v[rt // 8, pl.ds(16 * (rt % 8), 16)] = total
            return 0
        lax.fori_loop(0, _CH, _row, 0)

        for g in range(_CH // 16):
            gt = ck * (_CH // 16) + g
            ys = y_v[pl.ds(ck * _CH + g * 16, 16)]
            rows = lane + g * 16
            vals = plsc.load_gather(buf, [rows, ys])
            picked_v[pl.ds(gt * 16, 16)] = vals
            plsc.addupdate_scatter(hist_v, [ys + lane_off], ones16i)

    _issue(0, buf_a, sem_a)
    _issue(1, buf_b, sem_b)

    def _outer(k2, _):
        for b, buf, sem in ((0, buf_a, sem_a), (1, buf_b, sem_b)):
            ck = 2 * k2 + b
            _drain(buf, sem)
            _process(ck, buf)
            @pl.when(ck + 2 < _NCHUNK)
            def _():
                _issue(ck + 2, buf, sem)
        return 0
    lax.fori_loop(0, _NCHUNK // 2, _outer, 0)

    def _redzero(b, _):
        red_v[b // 8, pl.ds(16 * (b % 8), 16)] = jnp.zeros((16,), jnp.int32)
        return 0
    lax.fori_loop(0, 1024 // 16, _redzero, 0)

    def _red(b, _):
        acc = jnp.zeros((16,), jnp.int32)
        for l in range(16):
            acc = acc + hist_v[pl.ds(l * _HB + b * 16, 16)]
        red_v[b // 8, pl.ds(16 * (b % 8), 16)] = acc
        return 0
    lax.fori_loop(0, _HB // 16, _red, 0)

    pltpu.sync_copy(acc_v, sp_hbm.at[pl.ds(wid * (_RPW // 8), _RPW // 8)])
    pltpu.sync_copy(picked_v, picked_hbm.at[pl.ds(wid * _RPW, _RPW)])
    pltpu.sync_copy(red_v, hist_hbm.at[pl.ds(wid * 8, 8)])


_sc_pass = functools.partial(
    pl.kernel,
    out_type=[
        jax.ShapeDtypeStruct((_N // 8, 128), jnp.float32),
        jax.ShapeDtypeStruct((_N,), jnp.float32),
        jax.ShapeDtypeStruct((_NW * 8, 128), jnp.int32),
    ],
    mesh=plsc.VectorSubcoreMesh(core_axis_name="c", subcore_axis_name="s"),
    compiler_params=pltpu.CompilerParams(needs_layout_passes=False),
    scratch_types=[
        pltpu.VMEM((_RPW,), jnp.int32),
        pltpu.VMEM((_CH, _C), jnp.float32),
        pltpu.VMEM((_CH, _C), jnp.float32),
        pltpu.VMEM((_RPW // 8, 128), jnp.float32),
        pltpu.VMEM((_RPW,), jnp.float32),
        pltpu.VMEM((16 * _HB,), jnp.int32),
        pltpu.VMEM((8, 128), jnp.int32),
        pltpu.SemaphoreType.DMA,
        pltpu.SemaphoreType.DMA,
    ],
)(_sc_body)


def _combine_body(sp_ref, picked_ref, hist_ref, loss_ref):
    sp = sp_ref[...]
    col = lax.broadcasted_iota(jnp.int32, (128, 8), 0)
    grp = lax.broadcasted_iota(jnp.int32, (128, 8), 1)
    m = (col // 16 == grp).astype(jnp.float32)
    s8 = jax.lax.dot_general(sp, m, (((1,), (0,)), ((), ())),
                             preferred_element_type=jnp.float32)
    slog_s = jnp.sum(jnp.log(s8))
    slog_p = jnp.sum(jnp.log(picked_ref[...]))
    hist = hist_ref[...].reshape(_NW, 8, 128).astype(jnp.float32)
    cnt = jnp.sum(hist, axis=0)
    cterm = jnp.sum(cnt * jnp.log(jnp.maximum(cnt, 1.0)))
    nf = jnp.float32(_N)
    loss_ref[0, 0] = jnp.log(nf) + (slog_s - slog_p - cterm) / nf


@jax.jit
def _arb_loss(output, y):
    sp, picked, hist = _sc_pass(output, y.astype(jnp.int32))
    out = pl.pallas_call(
        _combine_body,
        out_specs=pl.BlockSpec(memory_space=pltpu.SMEM),
        out_shape=jax.ShapeDtypeStruct((1, 1), jnp.float32),
    )(sp, picked.reshape(128, 128), hist)
    return out.reshape(())


def kernel(output, y):
    return _arb_loss(output, y)
